# baseline (speedup 1.0000x reference)
import functools

import jax
import jax.numpy as jnp
from jax import lax
from jax.experimental import pallas as pl
from jax.experimental.pallas import tpu as pltpu
from jax.experimental.pallas import tpu_sc as plsc

_NC = 2
_NS = 16
_NW = _NC * _NS
_LANES = 16
_CH = 16
_B_SC = 1


def _sc_body(S, D, x_row0, x_hbm, pos_hbm, out_hbm, pb, xb, si0, si1, so0, so1):
    wid = lax.axis_index("s") * _NC + lax.axis_index("c")
    s_per_w = S // _NW
    nchunk = s_per_w // _CH
    base = wid * s_per_w
    sin = (si0, si1)
    sout = (so0, so1)

    def start_in(c):
        par = c % 2
        s0 = base + c * _CH
        hs = [pltpu.make_async_copy(pos_hbm.at[pl.ds(s0, _CH)],
                                    pb.at[par], sin[par]),
              pltpu.make_async_copy(x_hbm.at[pl.ds(x_row0 + s0, _CH)],
                                    xb.at[par], sin[par])]
        for h in hs:
            h.start()
        return hs

    def start_out(c):
        par = c % 2
        s0 = base + c * _CH
        h = pltpu.make_async_copy(xb.at[par], out_hbm.at[pl.ds(s0, _CH)],
                                  sout[par])
        h.start()
        return [h]

    def compute(par):
        def row(i, carry):
            @plsc.parallel_loop(0, D // _LANES, unroll=8)
            def col(j):
                sl = pl.ds(j * _LANES, _LANES)
                xb[par, i, sl] = xb[par, i, sl] + pb[par, i, sl]
            return carry
        lax.fori_loop(0, _CH, row, 0)

    in_h = {0: start_in(0)}
    out_h = {}
    for c in range(nchunk):
        if c + 1 < nchunk:
            if c - 1 >= 0:
                for h in out_h.pop(c - 1):
                    h.wait()
            in_h[c + 1] = start_in(c + 1)
        for h in in_h.pop(c):
            h.wait()
        compute(c % 2)
        out_h[c] = start_out(c)
    for c in (nchunk - 2, nchunk - 1):
        if c >= 0 and c in out_h:
            for h in out_h.pop(c):
                h.wait()


def _tc_add(x_ref, p_ref, o_ref):
    o_ref[...] = x_ref[...] + p_ref[...]


def kernel(x, pos_table):
    B, S, D = x.shape
    b_tc = B - _B_SC

    mesh = plsc.VectorSubcoreMesh(core_axis_name="c", subcore_axis_name="s")
    sc_add = pl.kernel(
        functools.partial(_sc_body, S, D, b_tc * S),
        out_type=jax.ShapeDtypeStruct((S, D), jnp.float32),
        mesh=mesh,
        scratch_types=[
            pltpu.VMEM((2, _CH, D), jnp.float32),
            pltpu.VMEM((2, _CH, D), jnp.float32),
            pltpu.SemaphoreType.DMA,
            pltpu.SemaphoreType.DMA,
            pltpu.SemaphoreType.DMA,
            pltpu.SemaphoreType.DMA,
        ],
    )
    sc_out = sc_add(x.reshape(B * S, D), pos_table)

    BS = 512
    tc_out = pl.pallas_call(
        _tc_add,
        grid=(S // BS, b_tc),
        in_specs=[
            pl.BlockSpec((1, BS, D), lambda j, b: (b, j, 0)),
            pl.BlockSpec((BS, D), lambda j, b: (j, 0)),
        ],
        out_specs=pl.BlockSpec((1, BS, D), lambda j, b: (b, j, 0)),
        out_shape=jax.ShapeDtypeStruct((b_tc, S, D), x.dtype),
    )(x, pos_table)

    return jnp.concatenate([tc_out, sc_out.reshape(1, S, D)], axis=0)


# speedup vs baseline: 1.4699x; 1.4699x over previous
import functools

import jax
import jax.numpy as jnp
from jax import lax
from jax.experimental import pallas as pl
from jax.experimental.pallas import tpu as pltpu
from jax.experimental.pallas import tpu_sc as plsc

_NC = 2
_NS = 16
_NW = _NC * _NS
_LANES = 16
_CH = 8


def _sc_body(B, S, D, x_hbm, pos_hbm, out_hbm,
             pb, xb0, xb1, xb2, xb3, si0, si1, so0, so1):
    wid = lax.axis_index("s") * _NC + lax.axis_index("c")
    s_per_w = S // _NW
    nchunk = s_per_w // _CH
    base = wid * s_per_w
    xbs = (xb0, xb1, xb2, xb3)
    sin = (si0, si1)
    sout = (so0, so1)

    def start_in(c):
        par = c % 2
        s0 = base + c * _CH
        hs = [pltpu.make_async_copy(pos_hbm.at[pl.ds(s0, _CH)],
                                    pb.at[par], sin[par])]
        for b in range(B):
            hs.append(pltpu.make_async_copy(x_hbm.at[pl.ds(b * S + s0, _CH)],
                                            xbs[b].at[par], sin[par]))
        for h in hs:
            h.start()
        return hs

    def start_out(c):
        par = c % 2
        s0 = base + c * _CH
        hs = []
        for b in range(B):
            hs.append(pltpu.make_async_copy(xbs[b].at[par],
                                            out_hbm.at[pl.ds(b * S + s0, _CH)],
                                            sout[par]))
        for h in hs:
            h.start()
        return hs

    ncol = D // _LANES

    def compute(par):
        @plsc.parallel_loop(0, _CH * ncol, unroll=8)
        def slice_add(j):
            i = j // ncol
            col = j % ncol
            sl = pl.ds(col * _LANES, _LANES)
            pv = pb[par, i, sl]
            for b in range(B):
                plsc.addupdate(xbs[b].at[par, i, sl], pv)

    in_h = {0: start_in(0)}
    out_h = {}
    for c in range(nchunk):
        if c + 1 < nchunk:
            if c - 1 >= 0:
                for h in out_h.pop(c - 1):
                    h.wait()
            in_h[c + 1] = start_in(c + 1)
        for h in in_h.pop(c):
            h.wait()
        compute(c % 2)
        out_h[c] = start_out(c)
    for c in (nchunk - 2, nchunk - 1):
        if c >= 0 and c in out_h:
            for h in out_h.pop(c):
                h.wait()


def kernel(x, pos_table):
    B, S, D = x.shape
    xf = x.reshape(B * S, D)

    mesh = plsc.VectorSubcoreMesh(core_axis_name="c", subcore_axis_name="s")
    sc_add = pl.kernel(
        functools.partial(_sc_body, B, S, D),
        out_type=jax.ShapeDtypeStruct((B * S, D), jnp.float32),
        mesh=mesh,
        scratch_types=[
            pltpu.VMEM((2, _CH, D), jnp.float32),
            pltpu.VMEM((2, _CH, D), jnp.float32),
            pltpu.VMEM((2, _CH, D), jnp.float32),
            pltpu.VMEM((2, _CH, D), jnp.float32),
            pltpu.VMEM((2, _CH, D), jnp.float32),
            pltpu.SemaphoreType.DMA,
            pltpu.SemaphoreType.DMA,
            pltpu.SemaphoreType.DMA,
            pltpu.SemaphoreType.DMA,
        ],
    )
    out = sc_add(xf, pos_table)
    return out.reshape(B, S, D)
